# baseline (speedup 1.0000x reference)
import functools

import jax
import jax.numpy as jnp
from jax import lax
from jax.experimental import pallas as pl
from jax.experimental.pallas import tpu as pltpu
from jax.experimental.pallas import tpu_sc as plsc

ALPHA = 0.1
KSTEPS = 5

NC = 2
NS = 16
NW = NC * NS
CB = 128
DW = 16


HALVES = 2


def _scatter_rows_kernel(NP, D, n_chunks):
  rpt = NP // NS
  reps = rpt // CB
  nh = n_chunks // HALVES
  assert n_chunks % HALVES == 0 and nh >= 3

  mesh = plsc.VectorSubcoreMesh(core_axis_name="c", subcore_axis_name="s")

  @functools.partial(
      pl.kernel,
      out_type=jax.ShapeDtypeStruct((NC, NP, D), jnp.float32),
      mesh=mesh,
      scratch_types=(
          [pltpu.VMEM((nh, CB), jnp.int32)] * 2
          + [pltpu.VMEM((CB, D), jnp.float32)] * 2
          + [pltpu.VMEM_SHARED((NP, D), jnp.float32)]
          + [pltpu.SemaphoreType.DMA] * 4
      ),
  )
  def step(g_hbm, src_hbm, dst_hbm, z_hbm, out_hbm, src_v, dst_v,
           r0, r1, acc_sh, gs0, gs1, ss0, ss1):
    bufs = (r0, r1)
    gs = (gs0, gs1)
    ss = (ss0, ss1)
    c = lax.axis_index("c")
    s = lax.axis_index("s")
    wid = c * NS + s
    pltpu.sync_copy(z_hbm, r0)

    def zbody(i, carry):
      pltpu.sync_copy(r0, acc_sh.at[pl.ds(s * rpt + i * CB, CB)])
      return carry

    lax.fori_loop(0, reps, zbody, 0)
    plsc.subcore_barrier()

    def fire_g(j, b):
      pltpu.async_copy(g_hbm.at[src_v.at[j]], bufs[b], gs[b])

    def wait_g(b):
      pltpu.make_async_copy(g_hbm.at[src_v.at[0]], bufs[b], gs[b]).wait()

    def fire_s(j, b):
      pltpu.async_copy(bufs[b], acc_sh.at[dst_v.at[j]], ss[b], add=True)

    def wait_s(b):
      pltpu.make_async_copy(bufs[b], acc_sh.at[dst_v.at[0]], ss[b]).wait()

    for h in range(HALVES):
      pltpu.sync_copy(src_hbm.at[wid, pl.ds(h * nh, nh)], src_v)
      pltpu.sync_copy(dst_hbm.at[wid, pl.ds(h * nh, nh)], dst_v)
      fire_g(0, 0)
      wait_g(0)
      fire_s(0, 0)
      fire_g(1, 1)

      def pbody(g, carry):
        j1 = 2 * g + 1
        wait_g(1)
        fire_s(j1, 1)
        wait_s(0)
        fire_g(j1 + 1, 0)
        wait_g(0)
        fire_s(j1 + 1, 0)
        wait_s(1)
        fire_g(j1 + 2, 1)
        return carry

      lax.fori_loop(0, (nh - 2) // 2, pbody, 0)
      wait_g(1)
      fire_s(nh - 1, 1)
      wait_s(0)
      wait_s(1)

    plsc.subcore_barrier()
    pltpu.sync_copy(acc_sh.at[pl.ds(s * rpt, rpt)],
                    out_hbm.at[c, pl.ds(s * rpt, rpt)])

  return step


def _degree_kernel(NP, n_chunks):
  rpt = NP // NS
  reps = rpt // CB

  mesh = plsc.VectorSubcoreMesh(core_axis_name="c", subcore_axis_name="s")

  @functools.partial(
      pl.kernel,
      out_type=jax.ShapeDtypeStruct((NC, NP, DW), jnp.float32),
      mesh=mesh,
      scratch_types=[
          pltpu.VMEM((n_chunks, CB), jnp.int32),
          pltpu.VMEM((CB, DW), jnp.float32),
          pltpu.VMEM_SHARED((NP, DW), jnp.float32),
          pltpu.SemaphoreType.DMA,
      ],
  )
  def deg(dst_hbm, zo_hbm, out_hbm, dst_v, buf_v, acc_sh, sem):
    c = lax.axis_index("c")
    s = lax.axis_index("s")
    wid = c * NS + s
    pltpu.sync_copy(dst_hbm.at[wid], dst_v)
    pltpu.sync_copy(zo_hbm.at[0], buf_v)

    def zbody(i, carry):
      pltpu.sync_copy(buf_v, acc_sh.at[pl.ds(s * rpt + i * CB, CB)])
      return carry

    lax.fori_loop(0, reps, zbody, 0)
    pltpu.sync_copy(zo_hbm.at[1], buf_v)
    plsc.subcore_barrier()

    W = min(8, n_chunks)

    def fire(j, carry):
      pltpu.async_copy(buf_v, acc_sh.at[dst_v.at[j]], sem, add=True)
      return carry

    def wait1(j, carry):
      pltpu.make_async_copy(buf_v, acc_sh.at[dst_v.at[0]], sem).wait()
      return carry

    def steady(j, carry):
      wait1(j, carry)
      return fire(j, carry)

    lax.fori_loop(0, W, fire, 0)
    lax.fori_loop(W, n_chunks, steady, 0)
    lax.fori_loop(0, W, wait1, 0)
    plsc.subcore_barrier()
    pltpu.sync_copy(acc_sh.at[pl.ds(s * rpt, rpt)],
                    out_hbm.at[c, pl.ds(s * rpt, rpt)])

  return deg


def _prep_call(degpair, x_pad, N, NP, D, RB):
  def body(deg_ref, x_ref, dis_ref, g0_ref):
    i = pl.program_id(0)
    cnt = deg_ref[0, :, 0:1] + deg_ref[1, :, 0:1] + 1.0
    dis = lax.rsqrt(cnt)
    row = lax.broadcasted_iota(jnp.int32, (RB, 1), 0) + i * RB
    dis = jnp.where(row < N, dis, 0.0)
    disb = jnp.broadcast_to(dis, (RB, D))
    dis_ref[...] = disb
    g0_ref[...] = disb * x_ref[...]

  return pl.pallas_call(
      body,
      grid=(NP // RB,),
      in_specs=[
          pl.BlockSpec((NC, RB, DW), lambda i: (0, i, 0)),
          pl.BlockSpec((RB, D), lambda i: (i, 0)),
      ],
      out_specs=[
          pl.BlockSpec((RB, D), lambda i: (i, 0)),
          pl.BlockSpec((RB, D), lambda i: (i, 0)),
      ],
      out_shape=[
          jax.ShapeDtypeStruct((NP, D), jnp.float32),
          jax.ShapeDtypeStruct((NP, D), jnp.float32),
      ],
  )(degpair, x_pad)


def _epi_call(accpair, g, dis, P, NP, D, RB):
  def body(acc_ref, g_ref, dis_ref, p_ref, gn_ref, pn_ref):
    d = dis_ref[...]
    h = d * (acc_ref[0] + acc_ref[1] + g_ref[...])
    pn_ref[...] = p_ref[...] + h
    gn_ref[...] = d * h

  return pl.pallas_call(
      body,
      grid=(NP // RB,),
      in_specs=[
          pl.BlockSpec((NC, RB, D), lambda i: (0, i, 0)),
          pl.BlockSpec((RB, D), lambda i: (i, 0)),
          pl.BlockSpec((RB, D), lambda i: (i, 0)),
          pl.BlockSpec((RB, D), lambda i: (i, 0)),
      ],
      out_specs=[
          pl.BlockSpec((RB, D), lambda i: (i, 0)),
          pl.BlockSpec((RB, D), lambda i: (i, 0)),
      ],
      out_shape=[
          jax.ShapeDtypeStruct((NP, D), jnp.float32),
          jax.ShapeDtypeStruct((NP, D), jnp.float32),
      ],
  )(accpair, g, dis, P)


def _final_call(accpair, g, dis, P, x, W, b2, N, D, RB):
  cs = (1.0 - ALPHA) / KSTEPS

  def body(acc_ref, g_ref, dis_ref, p_ref, x_ref, w_ref, b_ref, o_ref):
    h = dis_ref[...] * (acc_ref[0] + acc_ref[1] + g_ref[...])
    pre = ALPHA * x_ref[...] + cs * (p_ref[...] + h)
    o_ref[...] = lax.dot_general(
        pre, w_ref[...], (((1,), (1,)), ((), ())),
        preferred_element_type=jnp.float32) + b_ref[...]

  return pl.pallas_call(
      body,
      grid=(N // RB,),
      in_specs=[
          pl.BlockSpec((NC, RB, D), lambda i: (0, i, 0)),
          pl.BlockSpec((RB, D), lambda i: (i, 0)),
          pl.BlockSpec((RB, D), lambda i: (i, 0)),
          pl.BlockSpec((RB, D), lambda i: (i, 0)),
          pl.BlockSpec((RB, D), lambda i: (i, 0)),
          pl.BlockSpec((D, D), lambda i: (0, 0)),
          pl.BlockSpec((1, D), lambda i: (0, 0)),
      ],
      out_specs=pl.BlockSpec((RB, D), lambda i: (i, 0)),
      out_shape=jax.ShapeDtypeStruct((N, D), jnp.float32),
  )(accpair, g, dis, P, x, W, b2)


def kernel(x, edge_index, W, b):
  N, D = x.shape
  E = edge_index.shape[1]

  NP = -(-(N + 1) // (NS * CB)) * (NS * CB)
  n_chunks = -(-(-(-E // (NW * CB))) // 4) * 4
  epad = NW * n_chunks * CB

  src = edge_index[0]
  dst = edge_index[1]
  padi = N + (jnp.arange(epad - E, dtype=jnp.int32) % (NP - N))
  src_p = jnp.concatenate([src, padi]).reshape(NW, n_chunks, CB)
  dst_p = jnp.concatenate([dst, padi]).reshape(NW, n_chunks, CB)

  x_pad = jnp.pad(x, ((0, NP - N), (0, 0)))
  zrows = jnp.zeros((CB, D), jnp.float32)
  zo = jnp.stack([jnp.zeros((CB, DW), jnp.float32),
                  jnp.ones((CB, DW), jnp.float32)])

  deg_k = _degree_kernel(NP, n_chunks)
  step_k = _scatter_rows_kernel(NP, D, n_chunks)

  degpair = deg_k(dst_p, zo)
  RB = 1024
  dis, g = _prep_call(degpair, x_pad, N, NP, D, RB)

  P = jnp.zeros((NP, D), jnp.float32)
  for _ in range(KSTEPS - 1):
    accpair = step_k(g, src_p, dst_p, zrows)
    g, P = _epi_call(accpair, g, dis, P, NP, D, RB)

  accpair = step_k(g, src_p, dst_p, zrows)
  b2 = jnp.reshape(b, (1, D))
  return _final_call(accpair, g, dis, P, x, W, b2, N, D, 1000)


# speedup vs baseline: 1.0298x; 1.0298x over previous
import functools

import jax
import jax.numpy as jnp
from jax import lax
from jax.experimental import pallas as pl
from jax.experimental.pallas import tpu as pltpu
from jax.experimental.pallas import tpu_sc as plsc

ALPHA = 0.1
KSTEPS = 5

NC = 2
NS = 16
NW = NC * NS
CB = 64
DW = 16


HALVES = 4
NB = 4
LA = 2


def _scatter_rows_kernel(NP, D, n_chunks):
  rpt = NP // NS
  reps = rpt // CB
  nh = n_chunks // HALVES
  G = nh // NB
  assert n_chunks % HALVES == 0 and nh % NB == 0 and G >= 2

  mesh = plsc.VectorSubcoreMesh(core_axis_name="c", subcore_axis_name="s")

  @functools.partial(
      pl.kernel,
      out_type=jax.ShapeDtypeStruct((NC, NP, D), jnp.float32),
      mesh=mesh,
      scratch_types=(
          [pltpu.VMEM((nh, CB), jnp.int32)] * 2
          + [pltpu.VMEM((CB, D), jnp.float32)] * NB
          + [pltpu.VMEM_SHARED((NP, D), jnp.float32)]
          + [pltpu.SemaphoreType.DMA] * (2 * NB)
      ),
  )
  def step(g_hbm, src_hbm, dst_hbm, z_hbm, out_hbm, src_v, dst_v,
           r0, r1, r2, r3, acc_sh,
           gs0, gs1, gs2, gs3, ss0, ss1, ss2, ss3):
    bufs = (r0, r1, r2, r3)
    gs = (gs0, gs1, gs2, gs3)
    ss = (ss0, ss1, ss2, ss3)
    c = lax.axis_index("c")
    s = lax.axis_index("s")
    wid = c * NS + s
    pltpu.sync_copy(z_hbm, r0)

    def zbody(i, carry):
      pltpu.sync_copy(r0, acc_sh.at[pl.ds(s * rpt + i * CB, CB)])
      return carry

    lax.fori_loop(0, reps, zbody, 0)
    plsc.subcore_barrier()

    def fire_g(j, b):
      pltpu.async_copy(g_hbm.at[src_v.at[j]], bufs[b], gs[b])

    def wait_g(b):
      pltpu.make_async_copy(g_hbm.at[src_v.at[0]], bufs[b], gs[b]).wait()

    def fire_s(j, b):
      pltpu.async_copy(bufs[b], acc_sh.at[dst_v.at[j]], ss[b], add=True)

    def wait_s(b):
      pltpu.make_async_copy(bufs[b], acc_sh.at[dst_v.at[0]], ss[b]).wait()

    for h in range(HALVES):
      pltpu.sync_copy(src_hbm.at[wid, pl.ds(h * nh, nh)], src_v)
      pltpu.sync_copy(dst_hbm.at[wid, pl.ds(h * nh, nh)], dst_v)
      for b in range(LA):
        fire_g(b, b)
      for b in range(NB):
        wait_g(b)
        fire_s(b, b)
        bn = (b + LA) % NB
        if b + LA >= NB:
          wait_s(bn)
        fire_g(b + LA, bn)

      def gbody(g, carry):
        j0 = g * NB
        for b in range(NB):
          bn = (b + LA) % NB
          wait_g(b)
          fire_s(j0 + b, b)
          wait_s(bn)
          fire_g(j0 + b + LA, bn)
        return carry

      lax.fori_loop(1, G - 1, gbody, 0)

      j0 = (G - 1) * NB
      for b in range(NB):
        wait_g(b)
        fire_s(j0 + b, b)
        if b < NB - LA:
          bn = (b + LA) % NB
          wait_s(bn)
          fire_g(j0 + b + LA, bn)
      for b in range(NB):
        wait_s(b)

    plsc.subcore_barrier()
    pltpu.sync_copy(acc_sh.at[pl.ds(s * rpt, rpt)],
                    out_hbm.at[c, pl.ds(s * rpt, rpt)])

  return step


def _degree_kernel(NP, n_chunks):
  rpt = NP // NS
  reps = rpt // CB

  mesh = plsc.VectorSubcoreMesh(core_axis_name="c", subcore_axis_name="s")

  @functools.partial(
      pl.kernel,
      out_type=jax.ShapeDtypeStruct((NC, NP, DW), jnp.float32),
      mesh=mesh,
      scratch_types=[
          pltpu.VMEM((n_chunks, CB), jnp.int32),
          pltpu.VMEM((CB, DW), jnp.float32),
          pltpu.VMEM_SHARED((NP, DW), jnp.float32),
          pltpu.SemaphoreType.DMA,
      ],
  )
  def deg(dst_hbm, zo_hbm, out_hbm, dst_v, buf_v, acc_sh, sem):
    c = lax.axis_index("c")
    s = lax.axis_index("s")
    wid = c * NS + s
    pltpu.sync_copy(dst_hbm.at[wid], dst_v)
    pltpu.sync_copy(zo_hbm.at[0], buf_v)

    def zbody(i, carry):
      pltpu.sync_copy(buf_v, acc_sh.at[pl.ds(s * rpt + i * CB, CB)])
      return carry

    lax.fori_loop(0, reps, zbody, 0)
    pltpu.sync_copy(zo_hbm.at[1], buf_v)
    plsc.subcore_barrier()

    W = min(8, n_chunks)

    def fire(j, carry):
      pltpu.async_copy(buf_v, acc_sh.at[dst_v.at[j]], sem, add=True)
      return carry

    def wait1(j, carry):
      pltpu.make_async_copy(buf_v, acc_sh.at[dst_v.at[0]], sem).wait()
      return carry

    def steady(j, carry):
      wait1(j, carry)
      return fire(j, carry)

    lax.fori_loop(0, W, fire, 0)
    lax.fori_loop(W, n_chunks, steady, 0)
    lax.fori_loop(0, W, wait1, 0)
    plsc.subcore_barrier()
    pltpu.sync_copy(acc_sh.at[pl.ds(s * rpt, rpt)],
                    out_hbm.at[c, pl.ds(s * rpt, rpt)])

  return deg


def _prep_call(degpair, x_pad, N, NP, D, RB):
  def body(deg_ref, x_ref, dis_ref, g0_ref):
    i = pl.program_id(0)
    cnt = deg_ref[0, :, 0:1] + deg_ref[1, :, 0:1] + 1.0
    dis = lax.rsqrt(cnt)
    row = lax.broadcasted_iota(jnp.int32, (RB, 1), 0) + i * RB
    dis = jnp.where(row < N, dis, 0.0)
    disb = jnp.broadcast_to(dis, (RB, D))
    dis_ref[...] = disb
    g0_ref[...] = disb * x_ref[...]

  return pl.pallas_call(
      body,
      grid=(NP // RB,),
      in_specs=[
          pl.BlockSpec((NC, RB, DW), lambda i: (0, i, 0)),
          pl.BlockSpec((RB, D), lambda i: (i, 0)),
      ],
      out_specs=[
          pl.BlockSpec((RB, D), lambda i: (i, 0)),
          pl.BlockSpec((RB, D), lambda i: (i, 0)),
      ],
      out_shape=[
          jax.ShapeDtypeStruct((NP, D), jnp.float32),
          jax.ShapeDtypeStruct((NP, D), jnp.float32),
      ],
  )(degpair, x_pad)


def _epi_call(accpair, g, dis, P, NP, D, RB):
  def body(acc_ref, g_ref, dis_ref, p_ref, gn_ref, pn_ref):
    d = dis_ref[...]
    h = d * (acc_ref[0] + acc_ref[1] + g_ref[...])
    pn_ref[...] = p_ref[...] + h
    gn_ref[...] = d * h

  return pl.pallas_call(
      body,
      grid=(NP // RB,),
      in_specs=[
          pl.BlockSpec((NC, RB, D), lambda i: (0, i, 0)),
          pl.BlockSpec((RB, D), lambda i: (i, 0)),
          pl.BlockSpec((RB, D), lambda i: (i, 0)),
          pl.BlockSpec((RB, D), lambda i: (i, 0)),
      ],
      out_specs=[
          pl.BlockSpec((RB, D), lambda i: (i, 0)),
          pl.BlockSpec((RB, D), lambda i: (i, 0)),
      ],
      out_shape=[
          jax.ShapeDtypeStruct((NP, D), jnp.float32),
          jax.ShapeDtypeStruct((NP, D), jnp.float32),
      ],
  )(accpair, g, dis, P)


def _final_call(accpair, g, dis, P, x, W, b2, N, D, RB):
  cs = (1.0 - ALPHA) / KSTEPS

  def body(acc_ref, g_ref, dis_ref, p_ref, x_ref, w_ref, b_ref, o_ref):
    h = dis_ref[...] * (acc_ref[0] + acc_ref[1] + g_ref[...])
    pre = ALPHA * x_ref[...] + cs * (p_ref[...] + h)
    o_ref[...] = lax.dot_general(
        pre, w_ref[...], (((1,), (1,)), ((), ())),
        preferred_element_type=jnp.float32) + b_ref[...]

  return pl.pallas_call(
      body,
      grid=(N // RB,),
      in_specs=[
          pl.BlockSpec((NC, RB, D), lambda i: (0, i, 0)),
          pl.BlockSpec((RB, D), lambda i: (i, 0)),
          pl.BlockSpec((RB, D), lambda i: (i, 0)),
          pl.BlockSpec((RB, D), lambda i: (i, 0)),
          pl.BlockSpec((RB, D), lambda i: (i, 0)),
          pl.BlockSpec((D, D), lambda i: (0, 0)),
          pl.BlockSpec((1, D), lambda i: (0, 0)),
      ],
      out_specs=pl.BlockSpec((RB, D), lambda i: (i, 0)),
      out_shape=jax.ShapeDtypeStruct((N, D), jnp.float32),
  )(accpair, g, dis, P, x, W, b2)


def kernel(x, edge_index, W, b):
  N, D = x.shape
  E = edge_index.shape[1]

  NP = -(-(N + 1) // (NS * CB)) * (NS * CB)
  n_chunks = -(-(-(-E // (NW * CB))) // (HALVES * NB)) * (HALVES * NB)
  epad = NW * n_chunks * CB

  src = edge_index[0]
  dst = edge_index[1]
  padi = N + (jnp.arange(epad - E, dtype=jnp.int32) % (NP - N))
  src_p = jnp.concatenate([src, padi]).reshape(NW, n_chunks, CB)
  dst_p = jnp.concatenate([dst, padi]).reshape(NW, n_chunks, CB)

  x_pad = jnp.pad(x, ((0, NP - N), (0, 0)))
  zrows = jnp.zeros((CB, D), jnp.float32)
  zo = jnp.stack([jnp.zeros((CB, DW), jnp.float32),
                  jnp.ones((CB, DW), jnp.float32)])

  deg_k = _degree_kernel(NP, n_chunks)
  step_k = _scatter_rows_kernel(NP, D, n_chunks)

  degpair = deg_k(dst_p, zo)
  RB = 1024
  dis, g = _prep_call(degpair, x_pad, N, NP, D, RB)

  P = jnp.zeros((NP, D), jnp.float32)
  for _ in range(KSTEPS - 1):
    accpair = step_k(g, src_p, dst_p, zrows)
    g, P = _epi_call(accpair, g, dis, P, NP, D, RB)

  accpair = step_k(g, src_p, dst_p, zrows)
  b2 = jnp.reshape(b, (1, D))
  return _final_call(accpair, g, dis, P, x, W, b2, N, D, 1000)


# speedup vs baseline: 1.0326x; 1.0027x over previous
import functools

import jax
import jax.numpy as jnp
from jax import lax
from jax.experimental import pallas as pl
from jax.experimental.pallas import tpu as pltpu
from jax.experimental.pallas import tpu_sc as plsc

ALPHA = 0.1
KSTEPS = 5

NC = 2
NS = 16
NW = NC * NS
CB = 64
DW = 16


HALVES = 4
NB = 4
LA = 2


def _scatter_rows_kernel(NP, D, n_chunks):
  rpt = NP // NS
  reps = rpt // CB
  nh = n_chunks // HALVES
  G = nh // NB
  assert n_chunks % HALVES == 0 and nh % NB == 0 and G >= 2

  mesh = plsc.VectorSubcoreMesh(core_axis_name="c", subcore_axis_name="s")

  @functools.partial(
      pl.kernel,
      out_type=jax.ShapeDtypeStruct((NC, NP, D), jnp.float32),
      mesh=mesh,
      scratch_types=(
          [pltpu.VMEM((nh, CB), jnp.int32)] * 2
          + [pltpu.VMEM((CB, D), jnp.float32)] * NB
          + [pltpu.VMEM_SHARED((NP, D), jnp.float32)]
          + [pltpu.SemaphoreType.DMA] * (2 * NB)
      ),
  )
  def step(g_hbm, src_hbm, dst_hbm, z_hbm, out_hbm, src_v, dst_v,
           r0, r1, r2, r3, acc_sh,
           gs0, gs1, gs2, gs3, ss0, ss1, ss2, ss3):
    bufs = (r0, r1, r2, r3)
    gs = (gs0, gs1, gs2, gs3)
    ss = (ss0, ss1, ss2, ss3)
    c = lax.axis_index("c")
    s = lax.axis_index("s")
    wid = c * NS + s
    pltpu.sync_copy(z_hbm, r0)

    def zbody(i, carry):
      pltpu.async_copy(r0, acc_sh.at[pl.ds(s * rpt + i * CB, CB)], gs0)
      return carry

    def zdrain(i, carry):
      pltpu.make_async_copy(r0, acc_sh.at[pl.ds(s * rpt, CB)], gs0).wait()
      return carry

    lax.fori_loop(0, reps, zbody, 0)
    lax.fori_loop(0, reps, zdrain, 0)
    plsc.subcore_barrier()

    def fire_g(j, b):
      pltpu.async_copy(g_hbm.at[src_v.at[j]], bufs[b], gs[b])

    def wait_g(b):
      pltpu.make_async_copy(g_hbm.at[src_v.at[0]], bufs[b], gs[b]).wait()

    def fire_s(j, b):
      pltpu.async_copy(bufs[b], acc_sh.at[dst_v.at[j]], ss[b], add=True)

    def wait_s(b):
      pltpu.make_async_copy(bufs[b], acc_sh.at[dst_v.at[0]], ss[b]).wait()

    for h in range(HALVES):
      pltpu.sync_copy(src_hbm.at[wid, pl.ds(h * nh, nh)], src_v)
      pltpu.sync_copy(dst_hbm.at[wid, pl.ds(h * nh, nh)], dst_v)
      for b in range(LA):
        fire_g(b, b)
      for b in range(NB):
        wait_g(b)
        fire_s(b, b)
        bn = (b + LA) % NB
        if b + LA >= NB:
          wait_s(bn)
        fire_g(b + LA, bn)

      def gbody(g, carry):
        j0 = g * NB
        for b in range(NB):
          bn = (b + LA) % NB
          wait_g(b)
          fire_s(j0 + b, b)
          wait_s(bn)
          fire_g(j0 + b + LA, bn)
        return carry

      lax.fori_loop(1, G - 1, gbody, 0)

      j0 = (G - 1) * NB
      for b in range(NB):
        wait_g(b)
        fire_s(j0 + b, b)
        if b < NB - LA:
          bn = (b + LA) % NB
          wait_s(bn)
          fire_g(j0 + b + LA, bn)
      for b in range(NB):
        wait_s(b)

    plsc.subcore_barrier()
    pltpu.sync_copy(acc_sh.at[pl.ds(s * rpt, rpt)],
                    out_hbm.at[c, pl.ds(s * rpt, rpt)])

  return step


def _degree_kernel(NP, n_chunks):
  rpt = NP // NS
  reps = rpt // CB

  mesh = plsc.VectorSubcoreMesh(core_axis_name="c", subcore_axis_name="s")

  @functools.partial(
      pl.kernel,
      out_type=jax.ShapeDtypeStruct((NC, NP, DW), jnp.float32),
      mesh=mesh,
      scratch_types=[
          pltpu.VMEM((n_chunks, CB), jnp.int32),
          pltpu.VMEM((CB, DW), jnp.float32),
          pltpu.VMEM_SHARED((NP, DW), jnp.float32),
          pltpu.SemaphoreType.DMA,
      ],
  )
  def deg(dst_hbm, zo_hbm, out_hbm, dst_v, buf_v, acc_sh, sem):
    c = lax.axis_index("c")
    s = lax.axis_index("s")
    wid = c * NS + s
    pltpu.sync_copy(dst_hbm.at[wid], dst_v)
    pltpu.sync_copy(zo_hbm.at[0], buf_v)

    def zbody(i, carry):
      pltpu.sync_copy(buf_v, acc_sh.at[pl.ds(s * rpt + i * CB, CB)])
      return carry

    lax.fori_loop(0, reps, zbody, 0)
    pltpu.sync_copy(zo_hbm.at[1], buf_v)
    plsc.subcore_barrier()

    W = min(8, n_chunks)

    def fire(j, carry):
      pltpu.async_copy(buf_v, acc_sh.at[dst_v.at[j]], sem, add=True)
      return carry

    def wait1(j, carry):
      pltpu.make_async_copy(buf_v, acc_sh.at[dst_v.at[0]], sem).wait()
      return carry

    def steady(j, carry):
      wait1(j, carry)
      return fire(j, carry)

    lax.fori_loop(0, W, fire, 0)
    lax.fori_loop(W, n_chunks, steady, 0)
    lax.fori_loop(0, W, wait1, 0)
    plsc.subcore_barrier()
    pltpu.sync_copy(acc_sh.at[pl.ds(s * rpt, rpt)],
                    out_hbm.at[c, pl.ds(s * rpt, rpt)])

  return deg


def _prep_call(degpair, x_pad, N, NP, D, RB):
  def body(deg_ref, x_ref, dis_ref, g0_ref):
    i = pl.program_id(0)
    cnt = deg_ref[0, :, 0:1] + deg_ref[1, :, 0:1] + 1.0
    dis = lax.rsqrt(cnt)
    row = lax.broadcasted_iota(jnp.int32, (RB, 1), 0) + i * RB
    dis = jnp.where(row < N, dis, 0.0)
    disb = jnp.broadcast_to(dis, (RB, D))
    dis_ref[...] = disb
    g0_ref[...] = disb * x_ref[...]

  return pl.pallas_call(
      body,
      grid=(NP // RB,),
      in_specs=[
          pl.BlockSpec((NC, RB, DW), lambda i: (0, i, 0)),
          pl.BlockSpec((RB, D), lambda i: (i, 0)),
      ],
      out_specs=[
          pl.BlockSpec((RB, D), lambda i: (i, 0)),
          pl.BlockSpec((RB, D), lambda i: (i, 0)),
      ],
      out_shape=[
          jax.ShapeDtypeStruct((NP, D), jnp.float32),
          jax.ShapeDtypeStruct((NP, D), jnp.float32),
      ],
  )(degpair, x_pad)


def _epi_call(accpair, g, dis, P, NP, D, RB):
  def body(acc_ref, g_ref, dis_ref, p_ref, gn_ref, pn_ref):
    d = dis_ref[...]
    h = d * (acc_ref[0] + acc_ref[1] + g_ref[...])
    pn_ref[...] = p_ref[...] + h
    gn_ref[...] = d * h

  return pl.pallas_call(
      body,
      grid=(NP // RB,),
      in_specs=[
          pl.BlockSpec((NC, RB, D), lambda i: (0, i, 0)),
          pl.BlockSpec((RB, D), lambda i: (i, 0)),
          pl.BlockSpec((RB, D), lambda i: (i, 0)),
          pl.BlockSpec((RB, D), lambda i: (i, 0)),
      ],
      out_specs=[
          pl.BlockSpec((RB, D), lambda i: (i, 0)),
          pl.BlockSpec((RB, D), lambda i: (i, 0)),
      ],
      out_shape=[
          jax.ShapeDtypeStruct((NP, D), jnp.float32),
          jax.ShapeDtypeStruct((NP, D), jnp.float32),
      ],
  )(accpair, g, dis, P)


def _final_call(accpair, g, dis, P, x, W, b2, N, D, RB):
  cs = (1.0 - ALPHA) / KSTEPS

  def body(acc_ref, g_ref, dis_ref, p_ref, x_ref, w_ref, b_ref, o_ref):
    h = dis_ref[...] * (acc_ref[0] + acc_ref[1] + g_ref[...])
    pre = ALPHA * x_ref[...] + cs * (p_ref[...] + h)
    o_ref[...] = lax.dot_general(
        pre, w_ref[...], (((1,), (1,)), ((), ())),
        preferred_element_type=jnp.float32) + b_ref[...]

  return pl.pallas_call(
      body,
      grid=(N // RB,),
      in_specs=[
          pl.BlockSpec((NC, RB, D), lambda i: (0, i, 0)),
          pl.BlockSpec((RB, D), lambda i: (i, 0)),
          pl.BlockSpec((RB, D), lambda i: (i, 0)),
          pl.BlockSpec((RB, D), lambda i: (i, 0)),
          pl.BlockSpec((RB, D), lambda i: (i, 0)),
          pl.BlockSpec((D, D), lambda i: (0, 0)),
          pl.BlockSpec((1, D), lambda i: (0, 0)),
      ],
      out_specs=pl.BlockSpec((RB, D), lambda i: (i, 0)),
      out_shape=jax.ShapeDtypeStruct((N, D), jnp.float32),
  )(accpair, g, dis, P, x, W, b2)


def kernel(x, edge_index, W, b):
  N, D = x.shape
  E = edge_index.shape[1]

  NP = -(-(N + 1) // (NS * CB)) * (NS * CB)
  n_chunks = -(-(-(-E // (NW * CB))) // (HALVES * NB)) * (HALVES * NB)
  epad = NW * n_chunks * CB

  src = edge_index[0]
  dst = edge_index[1]
  padi = N + (jnp.arange(epad - E, dtype=jnp.int32) % (NP - N))
  src_p = jnp.concatenate([src, padi]).reshape(NW, n_chunks, CB)
  dst_p = jnp.concatenate([dst, padi]).reshape(NW, n_chunks, CB)

  x_pad = jnp.pad(x, ((0, NP - N), (0, 0)))
  zrows = jnp.zeros((CB, D), jnp.float32)
  zo = jnp.stack([jnp.zeros((CB, DW), jnp.float32),
                  jnp.ones((CB, DW), jnp.float32)])

  deg_k = _degree_kernel(NP, n_chunks)
  step_k = _scatter_rows_kernel(NP, D, n_chunks)

  degpair = deg_k(dst_p, zo)
  RB = 1024
  dis, g = _prep_call(degpair, x_pad, N, NP, D, RB)

  P = jnp.zeros((NP, D), jnp.float32)
  for _ in range(KSTEPS - 1):
    accpair = step_k(g, src_p, dst_p, zrows)
    g, P = _epi_call(accpair, g, dis, P, NP, D, RB)

  accpair = step_k(g, src_p, dst_p, zrows)
  b2 = jnp.reshape(b, (1, D))
  return _final_call(accpair, g, dis, P, x, W, b2, N, D, 1000)
